# baseline (speedup 1.0000x reference)
import functools

import jax
import jax.numpy as jnp
from jax import lax
from jax.experimental import pallas as pl
from jax.experimental.pallas import tpu as pltpu
from jax.experimental.pallas import tpu_sc as plsc

N = 10000
D = 128
NC = 2
NS = 16
NW = NC * NS
CHUNK = 128
NCHUNK = 80
CPP = 40
E_PAD = NW * NCHUNK * CHUNK
ROWS_PER_TILE = 640
N_PAD = NS * ROWS_PER_TILE
ACC_PER_TILE = 632
ACC_ROWS = NS * ACC_PER_TILE
PERM_PER_W = N_PAD // NW
TB = 2000
TW = 128

def _sc_mesh():
    return plsc.VectorSubcoreMesh(core_axis_name="c", subcore_axis_name="s",
                                  num_cores=NC, num_subcores=NS)


def _worker_ids():
    c = lax.axis_index("c")
    s = lax.axis_index("s")
    return c, s, c * NS + s


GROWS = 80


def _newton_rsqrt(d):
    y = lax.bitcast_convert_type(
        jnp.int32(0x5F3759DF) - lax.shift_right_logical(
            lax.bitcast_convert_type(d, jnp.int32), 1), jnp.float32)
    for _ in range(3):
        y = y * (1.5 - 0.5 * d * y * y)
    return y


def _sc_fused_body(dst_hbm, src_hbm, x_hbm, y_hbm, perm_hbm,
                   deg_out, g_out, acc_out, accp_out, xp_out, degp_out,
                   yp_out,
                   deg_sp, y_sp, acc_sh, sidx_f, didx, buf0, buf1, dinv_v,
                   ones_v, zvec, pidx, sgv, sgi, sem0, sem1):
    c, s, wid = _worker_ids()
    pltpu.sync_copy(perm_hbm.at[pl.ds(s * ROWS_PER_TILE, ROWS_PER_TILE)],
                    pidx)

    @pl.when(s == 0)
    def _():
        pltpu.sync_copy(y_hbm, y_sp)

    def vinit(i, _):
        zvec[pl.ds(i * 16, 16)] = jnp.zeros((16,), jnp.float32)
        return 0

    lax.fori_loop(0, ROWS_PER_TILE // 16, vinit, 0)

    def oinit(i, _):
        ones_v[pl.ds(i * 16, 16)] = jnp.ones((16,), jnp.float32)
        return 0

    lax.fori_loop(0, CHUNK // 16, oinit, 0)

    def zinit(i, _):
        for k in range(D // 16):
            buf0[i, pl.ds(k * 16, 16)] = jnp.zeros((16,), jnp.float32)
        return 0

    lax.fori_loop(0, CHUNK, zinit, 0)
    pltpu.sync_copy(zvec, deg_sp.at[pl.ds(s * ROWS_PER_TILE, ROWS_PER_TILE)])
    for k in range(ACC_PER_TILE // CHUNK):
        pltpu.sync_copy(
            buf0, acc_sh.at[pl.ds(s * ACC_PER_TILE + k * CHUNK, CHUNK)])
    rem = ACC_PER_TILE % CHUNK
    if rem:
        pltpu.sync_copy(
            buf0.at[pl.ds(0, rem)],
            acc_sh.at[pl.ds(s * ACC_PER_TILE + ACC_PER_TILE - rem, rem)])
    plsc.subcore_barrier()

    def dchunk(j, _):
        pltpu.sync_copy(ones_v, deg_sp.at[didx.at[j]], add=True)
        return 0

    for k in range(2):
        for p in range(NCHUNK // CPP):
            pltpu.sync_copy(dst_hbm.at[2 * s + k, pl.ds(p * CPP, CPP)], didx)
            lax.fori_loop(0, CPP, dchunk, 0)
    plsc.subcore_barrier()

    pltpu.sync_copy(deg_sp.at[pl.ds(s * ROWS_PER_TILE, ROWS_PER_TILE)],
                    dinv_v)
    pltpu.sync_copy(deg_sp.at[pl.ds(s * ROWS_PER_TILE, ROWS_PER_TILE)],
                    deg_out.at[c, pl.ds(s * ROWS_PER_TILE, ROWS_PER_TILE)])

    def ninv(i, _):
        d = dinv_v[pl.ds(i * 16, 16)] + 1.0
        dinv_v[pl.ds(i * 16, 16)] = _newton_rsqrt(d)
        return 0

    lax.fori_loop(0, ROWS_PER_TILE // 16, ninv, 0)

    for k in range(ROWS_PER_TILE // GROWS):
        base = s * ROWS_PER_TILE + k * GROWS

        def grow(m, _, k=k):
            v16 = dinv_v[pl.ds(k * GROWS + m * 16, 16)]
            for rr in range(16):
                sc = v16[rr]
                for q in range(D // 16):
                    buf1[m * 16 + rr, pl.ds(q * 16, 16)] = (
                        buf1[m * 16 + rr, pl.ds(q * 16, 16)] * sc)
            return 0

        @pl.when(base < N)
        def _(base=base, grow=grow):
            pltpu.sync_copy(x_hbm.at[pl.ds(base, GROWS)],
                            buf1.at[pl.ds(0, GROWS)])
            lax.fori_loop(0, GROWS // 16, grow, 0)
            pltpu.sync_copy(buf1.at[pl.ds(0, GROWS)],
                            g_out.at[c, pl.ds(base, GROWS)])

    plsc.subcore_barrier()

    def _step(j, bufp, semp, bufq, semq):
        pltpu.make_async_copy(
            g_out.at[c].at[sidx_f.at[pl.ds(j * CHUNK, CHUNK)]],
            bufp, semp).wait()

        @pl.when(j + 1 < CPP)
        def _():
            pltpu.async_copy(
                g_out.at[c].at[sidx_f.at[pl.ds((j + 1) * CHUNK, CHUNK)]],
                bufq, semq)

        pltpu.sync_copy(bufp, acc_sh.at[didx.at[j]], add=True)

    def body(j, _):
        @pl.when(j % 2 == 0)
        def _():
            _step(j, buf0, sem0, buf1, sem1)

        @pl.when(j % 2 == 1)
        def _():
            _step(j, buf1, sem1, buf0, sem0)

        return 0

    for p in range(NCHUNK // CPP):
        pltpu.sync_copy(
            src_hbm.at[wid, pl.ds(p * CPP * CHUNK, CPP * CHUNK)], sidx_f)
        pltpu.sync_copy(dst_hbm.at[wid, pl.ds(p * CPP, CPP)], didx)
        pltpu.async_copy(
            g_out.at[c].at[sidx_f.at[pl.ds(0, CHUNK)]], buf0, sem0)
        lax.fori_loop(0, CPP, body, 0)
    plsc.subcore_barrier()
    pltpu.sync_copy(acc_sh.at[pl.ds(s * ACC_PER_TILE, ACC_PER_TILE)],
                    acc_out.at[c, pl.ds(s * ACC_PER_TILE, ACC_PER_TILE)])

    for k in range(ROWS_PER_TILE // CHUNK):
        po = s * ROWS_PER_TILE + k * CHUNK
        pltpu.async_copy(
            acc_sh.at[pidx.at[pl.ds(k * CHUNK, CHUNK)]], buf0, sem0).wait()
        pltpu.sync_copy(buf0, accp_out.at[c, pl.ds(po, CHUNK)])

        @pl.when(c == k % 2)
        def _(po=po, k=k):
            pltpu.async_copy(
                x_hbm.at[pidx.at[pl.ds(k * CHUNK, CHUNK)]],
                buf1, sem1).wait()
            pltpu.sync_copy(buf1, xp_out.at[pl.ds(po, CHUNK)])

    @pl.when(c == 0)
    def _():
        pltpu.async_copy(deg_sp.at[pidx], sgv, sem0).wait()
        pltpu.sync_copy(
            sgv, degp_out.at[pl.ds(s * ROWS_PER_TILE, ROWS_PER_TILE)])

    @pl.when(c == 1)
    def _():
        pltpu.async_copy(y_sp.at[pidx], sgi, sem1).wait()
        pltpu.sync_copy(
            sgi, yp_out.at[pl.ds(s * ROWS_PER_TILE, ROWS_PER_TILE)])


@functools.cache
def _fused_call():
    return pl.kernel(
        _sc_fused_body,
        out_type=[
            jax.ShapeDtypeStruct((NC, N_PAD), jnp.float32),
            jax.ShapeDtypeStruct((NC, N, D), jnp.float32),
            jax.ShapeDtypeStruct((NC, ACC_ROWS, D), jnp.float32),
            jax.ShapeDtypeStruct((NC, N_PAD, D), jnp.float32),
            jax.ShapeDtypeStruct((N_PAD, D), jnp.float32),
            jax.ShapeDtypeStruct((N_PAD,), jnp.float32),
            jax.ShapeDtypeStruct((N_PAD,), jnp.int32),
        ],
        mesh=_sc_mesh(),
        scratch_types=[
            pltpu.VMEM_SHARED((N_PAD,), jnp.float32),
            pltpu.VMEM_SHARED((N,), jnp.int32),
            pltpu.VMEM_SHARED((ACC_ROWS, D), jnp.float32),
            pltpu.VMEM((CPP * CHUNK,), jnp.int32),
            pltpu.VMEM((CPP, CHUNK), jnp.int32),
            pltpu.VMEM((CHUNK, D), jnp.float32),
            pltpu.VMEM((CHUNK, D), jnp.float32),
            pltpu.VMEM((ROWS_PER_TILE,), jnp.float32),
            pltpu.VMEM((CHUNK,), jnp.float32),
            pltpu.VMEM((ROWS_PER_TILE,), jnp.float32),
            pltpu.VMEM((ROWS_PER_TILE,), jnp.int32),
            pltpu.VMEM((ROWS_PER_TILE,), jnp.float32),
            pltpu.VMEM((ROWS_PER_TILE,), jnp.int32),
            pltpu.SemaphoreType.DMA,
            pltpu.SemaphoreType.DMA,
        ],
    )


def _lsm2(v):
    m = jnp.maximum(v[:, 0:1], v[:, 1:2])
    e0 = jnp.exp(v[:, 0:1] - m)
    e1 = jnp.exp(v[:, 1:2] - m)
    return v - m - jnp.log(e0 + e1)


def _tc_dense_body(acc_ref, x_ref, pt_ref, accp_ref, xp_ref, degp_ref,
                   yp_ref, tm_ref, wa_ref, ba_ref, wb_ref, bb_ref,
                   v1_ref, v2_ref, fb_ref,
                   poa_ref, pob_ref, paa_ref, pab_ref, augy_ref,
                   anm_ref, norm_ref):
    dinv = lax.rsqrt(pt_ref[...] + 1.0)
    z = dinv * (acc_ref[0] + acc_ref[1]) + (dinv * dinv) * x_ref[...]
    dinvp = lax.rsqrt(degp_ref[...] + 1.0)
    zp = (dinvp * (accp_ref[0] + accp_ref[1])
          + (dinvp * dinvp) * xp_ref[...])
    ha = jnp.maximum(
        jnp.dot(z, wa_ref[...], preferred_element_type=jnp.float32)
        + ba_ref[...], 0.0)
    hb = jnp.maximum(
        jnp.dot(z, wb_ref[...], preferred_element_type=jnp.float32)
        + bb_ref[...], 0.0)
    hbp = jnp.maximum(
        jnp.dot(zp, wb_ref[...], preferred_element_type=jnp.float32)
        + bb_ref[...], 0.0)
    l1 = jnp.dot(ha, v1_ref[...], preferred_element_type=jnp.float32) \
        + fb_ref[...]
    l2 = jnp.dot(hb, v2_ref[...], preferred_element_type=jnp.float32)
    l2p = jnp.dot(hbp, v2_ref[...], preferred_element_type=jnp.float32)
    lo = l1 + l2
    la = l1 + l2p
    poa_ref[...] = _lsm2(lo[:, 0:2])
    pob_ref[...] = _lsm2(lo[:, 2:4])
    paa_ref[...] = _lsm2(la[:, 0:2])
    pab_ref[...] = _lsm2(la[:, 2:4])
    augy = yp_ref[...]
    tm = tm_ref[...]
    augy_ref[...] = augy
    anm_ref[...] = jnp.where((augy == 1) & (tm != 0), 1, 0)
    norm_ref[...] = jnp.where((augy == 0) & (tm != 0), 1, 0)


_dense_call = pl.pallas_call(
    _tc_dense_body,
    grid=(N // TB,),
    in_specs=[
        pl.BlockSpec((NC, TB, D), lambda b: (0, b, 0)),
        pl.BlockSpec((TB, D), lambda b: (b, 0)),
        pl.BlockSpec((TB, 1), lambda b: (b, 0)),
        pl.BlockSpec((NC, TB, D), lambda b: (0, b, 0)),
        pl.BlockSpec((TB, D), lambda b: (b, 0)),
        pl.BlockSpec((TB, 1), lambda b: (b, 0)),
        pl.BlockSpec((TB, 1), lambda b: (b, 0)),
        pl.BlockSpec((TB, 1), lambda b: (b, 0)),
        pl.BlockSpec((D, D), lambda b: (0, 0)),
        pl.BlockSpec((1, D), lambda b: (0, 0)),
        pl.BlockSpec((D, D), lambda b: (0, 0)),
        pl.BlockSpec((1, D), lambda b: (0, 0)),
        pl.BlockSpec((D, 4), lambda b: (0, 0)),
        pl.BlockSpec((D, 4), lambda b: (0, 0)),
        pl.BlockSpec((1, 4), lambda b: (0, 0)),
    ],
    out_specs=[
        pl.BlockSpec((TB, 2), lambda b: (b, 0)),
        pl.BlockSpec((TB, 2), lambda b: (b, 0)),
        pl.BlockSpec((TB, 2), lambda b: (b, 0)),
        pl.BlockSpec((TB, 2), lambda b: (b, 0)),
        pl.BlockSpec((TB, 1), lambda b: (b, 0)),
        pl.BlockSpec((TB, 1), lambda b: (b, 0)),
        pl.BlockSpec((TB, 1), lambda b: (b, 0)),
    ],
    out_shape=[
        jax.ShapeDtypeStruct((N, 2), jnp.float32),
        jax.ShapeDtypeStruct((N, 2), jnp.float32),
        jax.ShapeDtypeStruct((N, 2), jnp.float32),
        jax.ShapeDtypeStruct((N, 2), jnp.float32),
        jax.ShapeDtypeStruct((N, 1), jnp.int32),
        jax.ShapeDtypeStruct((N, 1), jnp.int32),
        jax.ShapeDtypeStruct((N, 1), jnp.int32),
    ],
)


def kernel(x, edge_index, y, train_mask, val_mask, test_mask, perm,
           GNN_a_W, GNN_a_b, GNN_b_W, GNN_b_b,
           fc_a_W, fc_a_b, fc_b_W, fc_b_b):
    src = edge_index[0]
    dst = edge_index[1]
    e = src.shape[0]
    epw = e // NW
    ppw = NCHUNK * CHUNK - epw
    src_p = jnp.concatenate(
        [src.reshape(NW, epw), jnp.zeros((NW, ppw), jnp.int32)],
        axis=1).reshape(NW, NCHUNK, CHUNK)
    dummy = jnp.broadcast_to(
        N + (jnp.arange(ppw, dtype=jnp.int32) % (ACC_ROWS - N)), (NW, ppw))
    dst_p = jnp.concatenate(
        [dst.reshape(NW, epw), dummy], axis=1).reshape(NW, NCHUNK, CHUNK)

    perm_p = jnp.concatenate([perm, jnp.zeros((N_PAD - N,), jnp.int32)])
    degf, _g_scratch, acc, accp, xp, degperm, yperm = _fused_call()(
        dst_p, src_p.reshape(NW, NCHUNK * CHUNK), x, y, perm_p)
    pt = degf[0].reshape(N_PAD, 1)

    ba2 = GNN_a_b.reshape(1, D)
    bb2 = GNN_b_b.reshape(1, D)
    v1 = jnp.concatenate([fc_a_W[:D], fc_b_W[:D]], axis=1)
    v2 = jnp.concatenate([fc_a_W[D:], fc_b_W[D:]], axis=1)
    fb = jnp.concatenate([fc_a_b, fc_b_b]).reshape(1, 4)
    tm = train_mask.astype(jnp.int32).reshape(N, 1)

    poa, pob, paa, pab, augy, anm, nrm = _dense_call(
        acc, x, pt, accp, xp, degperm.reshape(N_PAD, 1),
        yperm.reshape(N_PAD, 1), tm, GNN_a_W, ba2, GNN_b_W, bb2,
        v1, v2, fb)

    return (poa, pob, paa, pab, augy[:, 0],
            train_mask, val_mask, test_mask,
            anm[:, 0].astype(bool), nrm[:, 0].astype(bool))


# speedup vs baseline: 1.0488x; 1.0488x over previous
import functools

import jax
import jax.numpy as jnp
from jax import lax
from jax.experimental import pallas as pl
from jax.experimental.pallas import tpu as pltpu
from jax.experimental.pallas import tpu_sc as plsc

N = 10000
D = 128
NC = 2
NS = 16
NW = NC * NS
CHUNK = 128
NCHUNK = 80
CPP = 40
E_PAD = NW * NCHUNK * CHUNK
ROWS_PER_TILE = 640
N_PAD = NS * ROWS_PER_TILE
ACC_PER_TILE = 632
ACC_ROWS = NS * ACC_PER_TILE
PERM_PER_W = N_PAD // NW
TB = 2000
TW = 128

def _sc_mesh():
    return plsc.VectorSubcoreMesh(core_axis_name="c", subcore_axis_name="s",
                                  num_cores=NC, num_subcores=NS)


def _worker_ids():
    c = lax.axis_index("c")
    s = lax.axis_index("s")
    return c, s, c * NS + s


GROWS = 80


def _newton_rsqrt(d):
    y = lax.bitcast_convert_type(
        jnp.int32(0x5F3759DF) - lax.shift_right_logical(
            lax.bitcast_convert_type(d, jnp.int32), 1), jnp.float32)
    for _ in range(3):
        y = y * (1.5 - 0.5 * d * y * y)
    return y


def _sc_fused_body(dst_hbm, src_hbm, x_hbm, deg_out, g_out, acc_out,
                   deg_sp, acc_sh, sidx_f, didx, buf0, buf1, dinv_v,
                   ones_v, zvec, sem0, sem1):
    c, s, wid = _worker_ids()

    def vinit(i, _):
        zvec[pl.ds(i * 16, 16)] = jnp.zeros((16,), jnp.float32)
        return 0

    lax.fori_loop(0, ROWS_PER_TILE // 16, vinit, 0)

    def oinit(i, _):
        ones_v[pl.ds(i * 16, 16)] = jnp.ones((16,), jnp.float32)
        return 0

    lax.fori_loop(0, CHUNK // 16, oinit, 0)

    def zinit(i, _):
        for k in range(D // 16):
            buf0[i, pl.ds(k * 16, 16)] = jnp.zeros((16,), jnp.float32)
        return 0

    lax.fori_loop(0, CHUNK, zinit, 0)
    pltpu.sync_copy(zvec, deg_sp.at[pl.ds(s * ROWS_PER_TILE, ROWS_PER_TILE)])
    for k in range(ACC_PER_TILE // CHUNK):
        pltpu.sync_copy(
            buf0, acc_sh.at[pl.ds(s * ACC_PER_TILE + k * CHUNK, CHUNK)])
    rem = ACC_PER_TILE % CHUNK
    if rem:
        pltpu.sync_copy(
            buf0.at[pl.ds(0, rem)],
            acc_sh.at[pl.ds(s * ACC_PER_TILE + ACC_PER_TILE - rem, rem)])
    plsc.subcore_barrier()

    def dchunk(j, _):
        pltpu.sync_copy(ones_v, deg_sp.at[didx.at[j]], add=True)
        return 0

    for k in range(2):
        for p in range(NCHUNK // CPP):
            pltpu.sync_copy(dst_hbm.at[2 * s + k, pl.ds(p * CPP, CPP)], didx)
            lax.fori_loop(0, CPP, dchunk, 0)
    plsc.subcore_barrier()

    pltpu.sync_copy(deg_sp.at[pl.ds(s * ROWS_PER_TILE, ROWS_PER_TILE)],
                    dinv_v)
    pltpu.sync_copy(deg_sp.at[pl.ds(s * ROWS_PER_TILE, ROWS_PER_TILE)],
                    deg_out.at[c, pl.ds(s * ROWS_PER_TILE, ROWS_PER_TILE)])

    def ninv(i, _):
        d = dinv_v[pl.ds(i * 16, 16)] + 1.0
        dinv_v[pl.ds(i * 16, 16)] = _newton_rsqrt(d)
        return 0

    lax.fori_loop(0, ROWS_PER_TILE // 16, ninv, 0)

    for k in range(ROWS_PER_TILE // GROWS):
        base = s * ROWS_PER_TILE + k * GROWS

        def grow(m, _, k=k):
            v16 = dinv_v[pl.ds(k * GROWS + m * 16, 16)]
            for rr in range(16):
                sc = v16[rr]
                for q in range(D // 16):
                    buf1[m * 16 + rr, pl.ds(q * 16, 16)] = (
                        buf1[m * 16 + rr, pl.ds(q * 16, 16)] * sc)
            return 0

        @pl.when(base < N)
        def _(base=base, grow=grow):
            pltpu.sync_copy(x_hbm.at[pl.ds(base, GROWS)],
                            buf1.at[pl.ds(0, GROWS)])
            lax.fori_loop(0, GROWS // 16, grow, 0)
            pltpu.sync_copy(buf1.at[pl.ds(0, GROWS)],
                            g_out.at[c, pl.ds(base, GROWS)])

    plsc.subcore_barrier()

    def _step(j, bufp, semp, bufq, semq):
        pltpu.make_async_copy(
            g_out.at[c].at[sidx_f.at[pl.ds(j * CHUNK, CHUNK)]],
            bufp, semp).wait()

        @pl.when(j + 1 < CPP)
        def _():
            pltpu.async_copy(
                g_out.at[c].at[sidx_f.at[pl.ds((j + 1) * CHUNK, CHUNK)]],
                bufq, semq)

        pltpu.sync_copy(bufp, acc_sh.at[didx.at[j]], add=True)

    def body(j, _):
        @pl.when(j % 2 == 0)
        def _():
            _step(j, buf0, sem0, buf1, sem1)

        @pl.when(j % 2 == 1)
        def _():
            _step(j, buf1, sem1, buf0, sem0)

        return 0

    for p in range(NCHUNK // CPP):
        pltpu.sync_copy(
            src_hbm.at[wid, pl.ds(p * CPP * CHUNK, CPP * CHUNK)], sidx_f)
        pltpu.sync_copy(dst_hbm.at[wid, pl.ds(p * CPP, CPP)], didx)
        pltpu.async_copy(
            g_out.at[c].at[sidx_f.at[pl.ds(0, CHUNK)]], buf0, sem0)
        lax.fori_loop(0, CPP, body, 0)
    plsc.subcore_barrier()
    pltpu.sync_copy(acc_sh.at[pl.ds(s * ACC_PER_TILE, ACC_PER_TILE)],
                    acc_out.at[c, pl.ds(s * ACC_PER_TILE, ACC_PER_TILE)])


@functools.cache
def _fused_call():
    return pl.kernel(
        _sc_fused_body,
        out_type=[
            jax.ShapeDtypeStruct((NC, N_PAD), jnp.float32),
            jax.ShapeDtypeStruct((NC, N, D), jnp.float32),
            jax.ShapeDtypeStruct((NC, ACC_ROWS, D), jnp.float32),
        ],
        mesh=_sc_mesh(),
        scratch_types=[
            pltpu.VMEM_SHARED((N_PAD,), jnp.float32),
            pltpu.VMEM_SHARED((ACC_ROWS, D), jnp.float32),
            pltpu.VMEM((CPP * CHUNK,), jnp.int32),
            pltpu.VMEM((CPP, CHUNK), jnp.int32),
            pltpu.VMEM((CHUNK, D), jnp.float32),
            pltpu.VMEM((CHUNK, D), jnp.float32),
            pltpu.VMEM((ROWS_PER_TILE,), jnp.float32),
            pltpu.VMEM((CHUNK,), jnp.float32),
            pltpu.VMEM((ROWS_PER_TILE,), jnp.float32),
            pltpu.SemaphoreType.DMA,
            pltpu.SemaphoreType.DMA,
        ],
    )


def _lsm2(v):
    m = jnp.maximum(v[:, 0:1], v[:, 1:2])
    e0 = jnp.exp(v[:, 0:1] - m)
    e1 = jnp.exp(v[:, 1:2] - m)
    return v - m - jnp.log(e0 + e1)


def _tc_main_body(acc_ref, x_ref, pt_ref, y_ref, wa_ref, ba_ref, wb_ref,
                  bb_ref, v1_ref, v2_ref, fb_ref,
                  poa_ref, pob_ref, l1f_ref, t_ref):
    ssum = pt_ref[...] + 1.0
    dinv = lax.rsqrt(ssum)
    a = acc_ref[0] + acc_ref[1]
    z = dinv * a + (dinv * dinv) * x_ref[...]
    ha = jnp.maximum(
        jnp.dot(z, wa_ref[...], preferred_element_type=jnp.float32)
        + ba_ref[...], 0.0)
    hb = jnp.maximum(
        jnp.dot(z, wb_ref[...], preferred_element_type=jnp.float32)
        + bb_ref[...], 0.0)
    l1 = jnp.dot(ha, v1_ref[...], preferred_element_type=jnp.float32) \
        + fb_ref[...]
    l2 = jnp.dot(hb, v2_ref[...], preferred_element_type=jnp.float32)
    lo = l1 + l2[:, 0:4]
    poa_ref[...] = _lsm2(lo[:, 0:2])
    pob_ref[...] = _lsm2(lo[:, 2:4])
    l1f_ref[...] = l1
    col = lax.broadcasted_iota(jnp.int32, (TB, TW), 1)
    t_ref[...] = jnp.where(col == 4, y_ref[...], l2)


_main_call = pl.pallas_call(
    _tc_main_body,
    grid=(N // TB,),
    in_specs=[
        pl.BlockSpec((NC, TB, D), lambda b: (0, b, 0)),
        pl.BlockSpec((TB, D), lambda b: (b, 0)),
        pl.BlockSpec((TB, 1), lambda b: (b, 0)),
        pl.BlockSpec((TB, 1), lambda b: (b, 0)),
        pl.BlockSpec((D, D), lambda b: (0, 0)),
        pl.BlockSpec((1, D), lambda b: (0, 0)),
        pl.BlockSpec((D, D), lambda b: (0, 0)),
        pl.BlockSpec((1, D), lambda b: (0, 0)),
        pl.BlockSpec((D, 4), lambda b: (0, 0)),
        pl.BlockSpec((D, TW), lambda b: (0, 0)),
        pl.BlockSpec((1, 4), lambda b: (0, 0)),
    ],
    out_specs=[
        pl.BlockSpec((TB, 2), lambda b: (b, 0)),
        pl.BlockSpec((TB, 2), lambda b: (b, 0)),
        pl.BlockSpec((TB, 4), lambda b: (b, 0)),
        pl.BlockSpec((TB, TW), lambda b: (b, 0)),
    ],
    out_shape=[
        jax.ShapeDtypeStruct((N, 2), jnp.float32),
        jax.ShapeDtypeStruct((N, 2), jnp.float32),
        jax.ShapeDtypeStruct((N, 4), jnp.float32),
        jax.ShapeDtypeStruct((N, TW), jnp.float32),
    ],
)


def _sc_perm_body(t_hbm, perm_hbm, tp_out, t_sp, pidx, rows, sem):
    c, s, wid = _worker_ids()
    base = wid * PERM_PER_W
    pltpu.sync_copy(perm_hbm.at[pl.ds(base, PERM_PER_W)], pidx)
    pltpu.sync_copy(t_hbm.at[pl.ds(s * 624, 624)],
                    t_sp.at[pl.ds(s * 624, 624)])

    @pl.when(s == NS - 1)
    def _():
        pltpu.sync_copy(t_hbm.at[pl.ds(NS * 624, N - NS * 624)],
                        t_sp.at[pl.ds(NS * 624, N - NS * 624)])

    plsc.subcore_barrier()
    pltpu.async_copy(t_sp.at[pidx], rows, sem).wait()
    pltpu.sync_copy(rows, tp_out.at[pl.ds(base, PERM_PER_W)])


@functools.cache
def _perm_call():
    return pl.kernel(
        _sc_perm_body,
        out_type=jax.ShapeDtypeStruct((N_PAD, TW), jnp.float32),
        mesh=_sc_mesh(),
        scratch_types=[
            pltpu.VMEM_SHARED((N, TW), jnp.float32),
            pltpu.VMEM((PERM_PER_W,), jnp.int32),
            pltpu.VMEM((PERM_PER_W, TW), jnp.float32),
            pltpu.SemaphoreType.DMA,
        ],
    )


def _tc_aug_body(l1f_ref, tp_ref, tm_ref, paa_ref, pab_ref, augy_ref,
                 anm_ref, norm_ref):
    v = l1f_ref[...] + tp_ref[:, 0:4]
    paa_ref[...] = _lsm2(v[:, 0:2])
    pab_ref[...] = _lsm2(v[:, 2:4])
    augy = tp_ref[:, 4:5].astype(jnp.int32)
    tm = tm_ref[...]
    augy_ref[...] = augy
    anm_ref[...] = jnp.where((augy == 1) & (tm != 0), 1, 0)
    norm_ref[...] = jnp.where((augy == 0) & (tm != 0), 1, 0)


_aug_call = pl.pallas_call(
    _tc_aug_body,
    grid=(N // TB,),
    in_specs=[
        pl.BlockSpec((TB, 4), lambda b: (b, 0)),
        pl.BlockSpec((TB, TW), lambda b: (b, 0)),
        pl.BlockSpec((TB, 1), lambda b: (b, 0)),
    ],
    out_specs=[
        pl.BlockSpec((TB, 2), lambda b: (b, 0)),
        pl.BlockSpec((TB, 2), lambda b: (b, 0)),
        pl.BlockSpec((TB, 1), lambda b: (b, 0)),
        pl.BlockSpec((TB, 1), lambda b: (b, 0)),
        pl.BlockSpec((TB, 1), lambda b: (b, 0)),
    ],
    out_shape=[
        jax.ShapeDtypeStruct((N, 2), jnp.float32),
        jax.ShapeDtypeStruct((N, 2), jnp.float32),
        jax.ShapeDtypeStruct((N, 1), jnp.int32),
        jax.ShapeDtypeStruct((N, 1), jnp.int32),
        jax.ShapeDtypeStruct((N, 1), jnp.int32),
    ],
)


def kernel(x, edge_index, y, train_mask, val_mask, test_mask, perm,
           GNN_a_W, GNN_a_b, GNN_b_W, GNN_b_b,
           fc_a_W, fc_a_b, fc_b_W, fc_b_b):
    src = edge_index[0]
    dst = edge_index[1]
    e = src.shape[0]
    epw = e // NW
    ppw = NCHUNK * CHUNK - epw
    src_p = jnp.concatenate(
        [src.reshape(NW, epw), jnp.zeros((NW, ppw), jnp.int32)],
        axis=1).reshape(NW, NCHUNK, CHUNK)
    dummy = jnp.broadcast_to(
        N + (jnp.arange(ppw, dtype=jnp.int32) % (ACC_ROWS - N)), (NW, ppw))
    dst_p = jnp.concatenate(
        [dst.reshape(NW, epw), dummy], axis=1).reshape(NW, NCHUNK, CHUNK)

    degp, _g_unused, acc = _fused_call()(
        dst_p, src_p.reshape(NW, NCHUNK * CHUNK), x)
    pt = degp[0].reshape(N_PAD, 1)

    ycol = y.astype(jnp.float32).reshape(N, 1)
    ba2 = GNN_a_b.reshape(1, D)
    bb2 = GNN_b_b.reshape(1, D)
    v1 = jnp.concatenate([fc_a_W[:D], fc_b_W[:D]], axis=1)
    v2 = jnp.concatenate(
        [fc_a_W[D:], fc_b_W[D:], jnp.zeros((D, TW - 4), jnp.float32)], axis=1)
    fb = jnp.concatenate([fc_a_b, fc_b_b]).reshape(1, 4)

    poa, pob, l1f, t = _main_call(acc, x, pt, ycol, GNN_a_W, ba2, GNN_b_W,
                                  bb2, v1, v2, fb)

    perm_p = jnp.concatenate([perm, jnp.zeros((N_PAD - N,), jnp.int32)])
    tp = _perm_call()(t, perm_p)

    tm = train_mask.astype(jnp.int32).reshape(N, 1)
    paa, pab, augy, anm, nrm = _aug_call(l1f, tp, tm)

    return (poa, pob, paa, pab, augy[:, 0],
            train_mask, val_mask, test_mask,
            anm[:, 0].astype(bool), nrm[:, 0].astype(bool))
